# Initial kernel scaffold; baseline (speedup 1.0000x reference)
import jax
import jax.numpy as jnp
from jax.experimental import pallas as pl


def kernel(inputs, W1, g1, b1, W2, g2, b2, W3, g3, b3, W4, g4, b4, W5, g5, b5, Wf, gf, bf, Wp1, gp1, bp1, Wp2, gp2, bp2, Wp3, bp3):
    raise NotImplementedError("write your pallas kernel here")


# speedup vs baseline: 5.7978x; 5.7978x over previous
import functools

import jax
import jax.numpy as jnp
from jax import lax
from jax.experimental import pallas as pl
from jax.experimental.pallas import tpu as pltpu
from jax.experimental.pallas import tpu_sc as plsc

B = 4
N = 4096
K = 20
CH = 64
BN = B * N
BNK = B * N * K
EPS = 1e-5
KPAD = 32
NEG_INF = float('-inf')


def _knn_body(rows_ref, full_ref, nrm_row_ref, nrm_col_ref, out_ref, *, br):
    b = pl.program_id(0)
    rows = rows_ref[0]
    full = full_ref[0]
    d = lax.dot_general(rows.astype(jnp.bfloat16), full.astype(jnp.bfloat16),
                        (((1,), (0,)), ((), ())),
                        preferred_element_type=jnp.float32)
    inner = -2.0 * d
    pd = ((-nrm_row_ref[0]) - inner) - nrm_col_ref[0]
    iota = lax.broadcasted_iota(jnp.int32, (br, N), 1)
    kiota = lax.broadcasted_iota(jnp.int32, (br, KPAD), 1)
    base = b * N

    def step(t, carry):
        pd_c, acc = carry
        m = jnp.max(pd_c, axis=1, keepdims=True)
        cand = jnp.where(pd_c == m, iota, jnp.int32(N))
        a = jnp.min(cand, axis=1, keepdims=True)
        acc = jnp.where(kiota == t, a + base, acc)
        pd_c = jnp.where(iota == a, NEG_INF, pd_c)
        return pd_c, acc

    _, acc = lax.fori_loop(0, K, step, (pd, jnp.zeros((br, KPAD), jnp.int32)),
                           unroll=True)
    out_ref[0] = acc


def _knn_flat_idx(pts, pts_cm, xx):
    _, _, C = pts.shape
    BR = 256
    nrm_row = xx[:, None, :]
    nrm_col = xx[:, :, None]
    return pl.pallas_call(
        functools.partial(_knn_body, br=BR),
        grid=(B, N // BR),
        in_specs=[
            pl.BlockSpec((1, BR, C), lambda b, i: (b, i, 0)),
            pl.BlockSpec((1, C, N), lambda b, i: (b, 0, 0)),
            pl.BlockSpec((1, 1, N), lambda b, i: (b, 0, 0)),
            pl.BlockSpec((1, BR, 1), lambda b, i: (b, i, 0)),
        ],
        out_specs=pl.BlockSpec((1, BR, KPAD), lambda b, i: (b, i, 0)),
        out_shape=jax.ShapeDtypeStruct((B, N, KPAD), jnp.int32),
    )(pts, pts_cm, nrm_row, nrm_col)


def _gather_rows(table, eidx, c):
    info = plsc.get_sparse_core_info()
    nw = info.num_cores * info.num_subcores
    per_w = BNK // nw
    chk = 128
    n_chunks = per_w // chk
    mesh = plsc.VectorSubcoreMesh(core_axis_name="c", subcore_axis_name="s")

    @functools.partial(
        pl.kernel, mesh=mesh,
        out_type=jax.ShapeDtypeStruct((BNK, c), jnp.float32),
        scratch_types=[
            pltpu.VMEM((per_w,), jnp.int32),
            pltpu.VMEM((chk, c), jnp.float32),
            pltpu.VMEM((chk, c), jnp.float32),
            pltpu.SemaphoreType.DMA,
            pltpu.SemaphoreType.DMA,
        ],
    )
    def k(table_hbm, idx_hbm, out_hbm, idx_v, buf0, buf1, sem0, sem1):
        wid = lax.axis_index("s") * info.num_cores + lax.axis_index("c")
        base = wid * per_w
        pltpu.sync_copy(idx_hbm.at[pl.ds(base, per_w)], idx_v)
        bufs = (buf0, buf1)
        sems = (sem0, sem1)
        for lane in (0, 1):
            pltpu.make_async_copy(
                table_hbm.at[idx_v.at[pl.ds(lane * chk, chk)]],
                bufs[lane], sems[lane]).start()

        def body(g2, _):
            g0 = g2 * 2
            for lane in (0, 1):
                g = g0 + lane
                pltpu.make_async_copy(
                    table_hbm.at[idx_v.at[pl.ds(g * chk, chk)]],
                    bufs[lane], sems[lane]).wait()
                pltpu.sync_copy(bufs[lane],
                                out_hbm.at[pl.ds(base + g * chk, chk)])
                @pl.when(g + 2 < n_chunks)
                def _():
                    pltpu.make_async_copy(
                        table_hbm.at[idx_v.at[pl.ds((g + 2) * chk, chk)]],
                        bufs[lane], sems[lane]).start()
            return ()

        lax.fori_loop(0, n_chunks // 2, body, (), unroll=False)

    return k(table, eidx)


def _mm_body(x_ref, w_ref, o_ref):
    o_ref[...] = lax.dot_general(
        x_ref[...].astype(jnp.bfloat16), w_ref[...].astype(jnp.bfloat16),
        (((1,), (1,)), ((), ())), preferred_element_type=jnp.float32)


def _mm(x, w, rb=2048):
    m, c = x.shape
    o = w.shape[0]
    return pl.pallas_call(
        _mm_body,
        grid=(m // rb,),
        in_specs=[pl.BlockSpec((rb, c), lambda i: (i, 0)),
                  pl.BlockSpec((o, c), lambda i: (0, 0))],
        out_specs=pl.BlockSpec((rb, o), lambda i: (i, 0)),
        out_shape=jax.ShapeDtypeStruct((m, o), jnp.float32),
    )(x, w)


def _mm_const_body(x_ref, w_ref, c_ref, o_ref):
    y = lax.dot_general(
        x_ref[0].astype(jnp.bfloat16), w_ref[...].astype(jnp.bfloat16),
        (((1,), (1,)), ((), ())), preferred_element_type=jnp.float32)
    o_ref[0] = y + c_ref[0]


def _mm_const(x, w, const, rb=2048):
    _, n, c = x.shape
    o = w.shape[0]
    return pl.pallas_call(
        _mm_const_body,
        grid=(B, n // rb),
        in_specs=[pl.BlockSpec((1, rb, c), lambda b, i: (b, i, 0)),
                  pl.BlockSpec((o, c), lambda b, i: (0, 0)),
                  pl.BlockSpec((1, 1, o), lambda b, i: (b, 0, 0))],
        out_specs=pl.BlockSpec((1, rb, o), lambda b, i: (b, i, 0)),
        out_shape=jax.ShapeDtypeStruct((B, n, o), jnp.float32),
    )(x, w, const)


def _mm_t_body(x_ref, w_ref, o_ref):
    o_ref[0] = lax.dot_general(
        w_ref[...].astype(jnp.bfloat16), x_ref[0].astype(jnp.bfloat16),
        (((1,), (1,)), ((), ())), preferred_element_type=jnp.float32)


def _mm_t(x, w, rb=5120):
    _, m, c = x.shape
    o = w.shape[0]
    return pl.pallas_call(
        _mm_t_body,
        grid=(B, m // rb),
        in_specs=[pl.BlockSpec((1, rb, c), lambda b, i: (b, i, 0)),
                  pl.BlockSpec((o, c), lambda b, i: (0, 0))],
        out_specs=pl.BlockSpec((1, o, rb), lambda b, i: (b, 0, i)),
        out_shape=jax.ShapeDtypeStruct((B, o, m), jnp.float32),
    )(x, w)


def _mm_cm_body(x_ref, w_ref, o_ref):
    o_ref[0] = lax.dot_general(
        w_ref[...].astype(jnp.bfloat16), x_ref[0].astype(jnp.bfloat16),
        (((1,), (0,)), ((), ())), preferred_element_type=jnp.float32)


def _mm_cm(x, w, rb=5120):
    _, c, m = x.shape
    o = w.shape[0]
    return pl.pallas_call(
        _mm_cm_body,
        grid=(B, m // rb),
        in_specs=[pl.BlockSpec((1, c, rb), lambda b, i: (b, 0, i)),
                  pl.BlockSpec((o, c), lambda b, i: (0, 0))],
        out_specs=pl.BlockSpec((1, o, rb), lambda b, i: (b, 0, i)),
        out_shape=jax.ShapeDtypeStruct((B, o, m), jnp.float32),
    )(x, w)


def _kmax_body(y_ref, o_ref):
    o_ref[0] = jnp.max(y_ref[0], axis=2)


def _kmax(y4):
    pb = 512
    return pl.pallas_call(
        _kmax_body,
        grid=(B, N // pb),
        in_specs=[pl.BlockSpec((1, CH, pb, K), lambda b, i: (b, 0, i, 0))],
        out_specs=pl.BlockSpec((1, CH, pb), lambda b, i: (b, 0, i)),
        out_shape=jax.ShapeDtypeStruct((B, CH, N), jnp.float32),
    )(y4)


def _bmax_body(y_ref, o_ref, acc_ref):
    i = pl.program_id(1)
    m = jnp.max(y_ref[0], axis=0, keepdims=True)
    @pl.when(i == 0)
    def _():
        acc_ref[...] = m
    @pl.when(i > 0)
    def _():
        acc_ref[...] = jnp.maximum(acc_ref[...], m)
    @pl.when(i == pl.num_programs(1) - 1)
    def _():
        o_ref[0] = acc_ref[...]


def _bmax(y, o):
    rb = 1024
    out = pl.pallas_call(
        _bmax_body,
        grid=(B, N // rb),
        in_specs=[pl.BlockSpec((1, rb, o), lambda b, i: (b, i, 0))],
        out_specs=pl.BlockSpec((1, 1, o), lambda b, i: (b, 0, 0)),
        out_shape=jax.ShapeDtypeStruct((B, 1, o), jnp.float32),
        scratch_shapes=[pltpu.VMEM((1, o), jnp.float32)],
    )(y)
    return out[:, 0, :]


def _bn_stats(x):
    m = jnp.mean(x, axis=0)
    v = jnp.var(x, axis=0)
    return m, v


def _bn_stats4(x4):
    return (jnp.mean(x4, axis=(0, 2, 3), keepdims=True),
            jnp.var(x4, axis=(0, 2, 3), keepdims=True))


def _bn_lrelu4(x, m, v, g, b):
    return jax.nn.leaky_relu(
        (x - m) / jnp.sqrt(v + EPS) * g[None, :, None, None]
        + b[None, :, None, None], 0.2)


def _bn_lrelu(x, m, v, g, b):
    return jax.nn.leaky_relu((x - m[None, :]) / jnp.sqrt(v[None, :] + EPS)
                             * g[None, :] + b[None, :], 0.2)


def _edge_stage(S, eidx, W, g, b, W2, g2, b2):
    c = S.shape[1]
    Sp = jnp.pad(S, ((0, 0), (0, 128 - c))) if c < 128 else S
    G = _gather_rows(Sp, eidx, 128)[:, :c]
    feat = G.reshape(B, N, K, c)
    xc = jnp.broadcast_to(S.reshape(B, N, 1, c), (B, N, K, c))
    gf = jnp.transpose(jnp.concatenate([feat - xc, xc], axis=3),
                       (0, 3, 1, 2))
    x = jnp.einsum('oc,bcnk->bonk', W, gf)
    m = jnp.mean(x, axis=(0, 2, 3), keepdims=True)
    v = jnp.var(x, axis=(0, 2, 3), keepdims=True)
    if W2 is None:
        M = _kmax(x)
        return jax.nn.leaky_relu(
            (M - m[:, :, :, 0]) / jnp.sqrt(v[:, :, :, 0] + EPS)
            * g[None, :, None] + b[None, :, None], 0.2)
    u = jax.nn.leaky_relu((x - m) / jnp.sqrt(v + EPS)
                          * g[None, :, None, None]
                          + b[None, :, None, None], 0.2)
    y = jnp.einsum('oc,bcnk->bonk', W2, u)
    m2 = jnp.mean(y, axis=(0, 2, 3), keepdims=True)
    v2 = jnp.var(y, axis=(0, 2, 3), keepdims=True)
    M = _kmax(y)
    return jax.nn.leaky_relu(
        (M - m2[:, :, :, 0]) / jnp.sqrt(v2[:, :, :, 0] + EPS)
        * g2[None, :, None] + b2[None, :, None], 0.2)


def kernel(inputs, W1, g1, b1, W2, g2, b2, W3, g3, b3, W4, g4, b4, W5, g5, b5,
           Wf, gf, bf, Wp1, gp1, bp1, Wp2, gp2, bp2, Wp3, bp3):
    pts0 = jnp.transpose(inputs, (0, 2, 1))
    S1 = pts0.reshape(BN, 9)

    xyz_cm = inputs[:, 0:3, :]
    xx1 = jnp.sum(xyz_cm * xyz_cm, axis=1)
    xyz = jnp.pad(pts0[:, :, :3], ((0, 0), (0, 0), (0, 5)))
    xyz_cmp = jnp.pad(xyz_cm, ((0, 0), (0, 5), (0, 0)))
    eidx1 = _knn_flat_idx(xyz, xyz_cmp, xx1)[:, :, :K].reshape(-1)
    net1 = _edge_stage(S1, eidx1, W1, g1, b1, W2, g2, b2)

    p2 = jnp.transpose(net1, (0, 2, 1))
    xx2 = jnp.sum(net1 * net1, axis=1)
    eidx2 = _knn_flat_idx(p2, net1, xx2)[:, :, :K].reshape(-1)
    net2 = _edge_stage(p2.reshape(BN, CH), eidx2, W3, g3, b3, W4, g4, b4)

    p3 = jnp.transpose(net2, (0, 2, 1))
    xx3 = jnp.sum(net2 * net2, axis=1)
    eidx3 = _knn_flat_idx(p3, net2, xx3)[:, :, :K].reshape(-1)
    net3 = _edge_stage(p3.reshape(BN, CH), eidx3, W5, g5, b5,
                       None, None, None)

    feats_cm = jnp.concatenate([net1, net2, net3], axis=1)
    feats = jnp.transpose(feats_cm, (0, 2, 1)).reshape(BN, 192)
    yf = _mm(feats, Wf)
    mf, vf = _bn_stats(yf)
    fmax = _bmax(yf.reshape(B, N, 1024), 1024)
    fusion = _bn_lrelu(fmax, mf, vf, gf, bf)

    const1 = _mm(jnp.pad(fusion, ((0, 4), (0, 0))), Wp1[:, :1024], rb=8)[:B]
    y1 = _mm_const(feats.reshape(B, N, 192), Wp1[:, 1024:],
                   const1[:, None, :]).reshape(BN, 512)
    mp1, vp1 = _bn_stats(y1)
    u1 = _bn_lrelu(y1, mp1, vp1, gp1, bp1)
    y2 = _mm(u1, Wp2)
    mp2, vp2 = _bn_stats(y2)
    u2 = _bn_lrelu(y2, mp2, vp2, gp2, bp2)
    Wp3p = jnp.pad(Wp3, ((0, 3), (0, 0)))
    y3 = _mm(u2, Wp3p)[:, :13] + bp3[None, :]
    return jnp.transpose(y3.reshape(B, N, 13), (0, 2, 1))
